# baseline (speedup 1.0000x reference)
import functools

import jax
import jax.numpy as jnp
from jax import lax
from jax.experimental import pallas as pl
from jax.experimental.pallas import tpu as pltpu
from jax.experimental.pallas import tpu_sc as plsc


def kernel(x, embedding):
    B, H = x.shape
    V, D = embedding.shape
    N = B * H

    info = plsc.get_sparse_core_info()
    NC, NS, L = info.num_cores, info.num_subcores, info.num_lanes
    NW = NC * NS

    K = 128
    NG = 3
    TL = 2
    NSB = 3
    DT, DI = D // 8, 8
    assert B == NW * K and D == DT * DI and V % 2 == 0
    RBYTES = K * D * 4

    CW = 256
    QC = CW // 2
    NFULL = V // CW
    TAILC = V - NFULL * CW
    STEPS = -(-NFULL // NW)
    PP = 136 * 65

    mesh = plsc.VectorSubcoreMesh(core_axis_name="c", subcore_axis_name="s")
    cparams = pltpu.CompilerParams(
        use_tc_tiling_on_sc=True, needs_layout_passes=False
    )

    xT = jnp.swapaxes(x, 0, 1).astype(jnp.int32)
    embT = jnp.swapaxes(embedding, 0, 1)
    tailT = jnp.swapaxes(embedding[NFULL * CW :, :], 0, 1) if TAILC else None

    @functools.partial(
        pl.kernel,
        out_type=jax.ShapeDtypeStruct((V // 2, 2 * D), jnp.float32),
        mesh=mesh,
        scratch_types=[
            pltpu.VMEM((NG, D, CW), jnp.float32),
            pltpu.VMEM((2, 136, 129), jnp.float32),
            pltpu.VMEM((D, max(TAILC, 1)), jnp.float32),
            pltpu.SemaphoreType.DMA((NG,)),
            pltpu.SemaphoreType.DMA((2,)),
        ],
        compiler_params=cparams,
    )
    def fmt_kernel(embT_hbm, tail_hbm, tbl_hbm, in_v, tout_v, tail_v, gsem, ssem):
        wid = lax.axis_index("s") * NC + lax.axis_index("c")

        lane = lax.iota(jnp.int32, L)
        halfl = lax.shift_right_logical(lane, 1)
        parl64 = jnp.left_shift(jnp.bitwise_and(lane, 1), 6)
        qqv = [halfl + 8 * g for g in range(CW // L)]

        def stage(s):
            g = s * NW + wid
            sg = s % NG

            @pl.when(g < NFULL)
            def _():
                pltpu.async_copy(
                    embT_hbm.at[:, pl.ds(g * CW, CW)], in_v.at[sg], gsem.at[sg]
                )

        def flush(s2, bs):
            @pl.when(s2 >= 2)
            def _():
                pltpu.make_async_copy(
                    tout_v.at[bs, pl.ds(0, QC), pl.ds(0, 2 * D)],
                    tbl_hbm.at[pl.ds(0, QC)],
                    ssem.at[bs],
                ).wait()

        def process(s):
            g = s * NW + wid
            bs = s % 2
            sg = s % NG

            @pl.when(g < NFULL)
            def _():
                flush(s, bs)
                pltpu.make_async_copy(
                    embT_hbm.at[:, pl.ds(g * CW, CW)], in_v.at[sg], gsem.at[sg]
                ).wait()
                src = in_v.at[sg]
                dst = tout_v.at[bs]

                @plsc.parallel_loop(0, D, unroll=2)
                def _(d):
                    jv = parl64 + d
                    for gi in range(CW // L):
                        vals = src[d, pl.ds(gi * L, L)]
                        plsc.store_scatter(dst, [qqv[gi], jv], vals)

                pltpu.async_copy(
                    tout_v.at[bs, pl.ds(0, QC), pl.ds(0, 2 * D)],
                    tbl_hbm.at[pl.ds(g * QC, QC)],
                    ssem.at[bs],
                )

        def body(s, carry):
            @pl.when(s < STEPS)
            def _():
                stage(s)

            @pl.when(s >= TL)
            def _():
                process(s - TL)

            return carry

        lax.fori_loop(0, STEPS + TL, body, 0)
        for bs in range(2):
            pltpu.make_async_copy(
                tout_v.at[bs, pl.ds(0, QC), pl.ds(0, 2 * D)],
                tbl_hbm.at[pl.ds(0, QC)],
                ssem.at[bs],
            ).wait()

        if TAILC:
            qt = TAILC // 2

            @pl.when(wid == 0)
            def _():
                pltpu.sync_copy(tail_hbm, tail_v)
                src = tail_v
                dst = tout_v.at[0]

                @plsc.parallel_loop(0, D, unroll=2)
                def _(d):
                    jv = parl64 + d
                    for gi in range(TAILC // L):
                        vals = src[d, pl.ds(gi * L, L)]
                        plsc.store_scatter(dst, [qqv[gi], jv], vals)

                pltpu.async_copy(
                    tout_v.at[0, pl.ds(0, qt), pl.ds(0, 2 * D)],
                    tbl_hbm.at[pl.ds(NFULL * QC, qt)],
                    ssem.at[0],
                )
                pltpu.make_async_copy(
                    tout_v.at[0, pl.ds(0, qt), pl.ds(0, 2 * D)],
                    tbl_hbm.at[pl.ds(0, qt)],
                    ssem.at[0],
                ).wait()

    @functools.partial(
        pl.kernel,
        out_type=jax.ShapeDtypeStruct((H, DT, NW, DI, K), jnp.float32),
        mesh=mesh,
        scratch_types=[
            pltpu.VMEM((H, K), jnp.int32),
            pltpu.VMEM((NG, K), jnp.int32),
            pltpu.VMEM((NG, K), jnp.int32),
            pltpu.VMEM((NG, K, 2 * D), jnp.float32),
            pltpu.VMEM((NSB, DT, DI, K + 1), jnp.float32),
            pltpu.SemaphoreType.DMA((NG,)),
            pltpu.SemaphoreType.DMA((NSB,)),
        ],
        compiler_params=cparams,
    )
    def emb_kernel(
        idx_hbm, tbl_hbm, out_hbm, idx_v, q_v, par_v, rows_v, tout_v, gsem, ssem
    ):
        wid = lax.axis_index("s") * NC + lax.axis_index("c")
        pltpu.sync_copy(idx_hbm.at[:, pl.ds(wid * K, K)], idx_v)

        lane = lax.iota(jnp.int32, L)
        dcols = [lane + d0 * L for d0 in range(D // L)]
        dtv = [lax.shift_right_logical(lane + d0 * L, 3) for d0 in range(D // L)]
        div = [jnp.bitwise_and(lane + d0 * L, DI - 1) for d0 in range(D // L)]

        def transpose_store(ht):
            bg = ht % NG
            bs = ht % NSB

            @pl.when(ht >= NSB)
            def _():
                pltpu.make_async_copy(
                    tout_v.at[bs, :, :, pl.ds(0, K)],
                    out_hbm.at[0, :, wid],
                    ssem.at[bs],
                ).wait()

            pltpu.make_async_copy(
                tbl_hbm.at[q_v.at[bg]], rows_v.at[bg], gsem.at[bg]
            ).wait()

            src = rows_v.at[bg]
            par = par_v.at[bg]
            dst = tout_v.at[bs]

            @plsc.parallel_loop(0, K, unroll=2)
            def _(b):
                col = jax.lax.broadcast(b, (L,))
                pv = plsc.load_gather(par, [col])
                base = jnp.left_shift(pv, 6)
                for d0 in range(D // L):
                    vals = plsc.load_gather(src, [col, base + dcols[d0]])
                    plsc.store_scatter(dst, [dtv[d0], div[d0], col], vals)

            pltpu.async_copy(
                tout_v.at[bs, :, :, pl.ds(0, K)],
                out_hbm.at[ht, :, wid],
                ssem.at[bs],
            )

        def body(h, carry):
            @pl.when(h < H)
            def _():
                bg = h % NG
                for i in range(K // L):
                    r = idx_v[h, pl.ds(i * L, L)]
                    q_v[bg, pl.ds(i * L, L)] = lax.shift_right_logical(r, 1)
                    par_v[bg, pl.ds(i * L, L)] = jnp.bitwise_and(r, 1)
                pltpu.async_copy(
                    tbl_hbm.at[q_v.at[bg]], rows_v.at[bg], gsem.at[bg]
                )

            @pl.when(h >= TL)
            def _():
                transpose_store(h - TL)

            return carry

        lax.fori_loop(0, H + TL, body, 0)
        for b in range(NSB):
            pltpu.make_async_copy(
                tout_v.at[b, :, :, pl.ds(0, K)], out_hbm.at[0, :, wid], ssem.at[b]
            ).wait()

    tbl2 = fmt_kernel(embT, tailT)
    out5 = emb_kernel(xT, tbl2)
    return out5.transpose(2, 4, 0, 1, 3).reshape(B, H, D)


# speedup vs baseline: 2.1648x; 2.1648x over previous
import functools

import jax
import jax.numpy as jnp
from jax import lax
from jax.experimental import pallas as pl
from jax.experimental.pallas import tpu as pltpu
from jax.experimental.pallas import tpu_sc as plsc


def kernel(x, embedding):
    B, H = x.shape
    V, D = embedding.shape
    N = B * H

    info = plsc.get_sparse_core_info()
    NC, NS, L = info.num_cores, info.num_subcores, info.num_lanes
    NW = NC * NS

    K = 128
    NG = 4
    TL = 2
    NSB = 4
    DT, DI = D // 8, 8
    assert B == NW * K and D == DT * DI
    RBYTES = K * D * 4

    xT = jnp.swapaxes(x, 0, 1).astype(jnp.int32)

    mesh = plsc.VectorSubcoreMesh(core_axis_name="c", subcore_axis_name="s")

    @functools.partial(
        pl.kernel,
        out_type=jax.ShapeDtypeStruct((H, DT, NW, DI, K), jnp.float32),
        mesh=mesh,
        scratch_types=[
            pltpu.VMEM((H, K), jnp.int32),
            pltpu.VMEM((NG, K, D), jnp.float32),
            pltpu.VMEM((NSB, DT, DI, K + 1), jnp.float32),
            pltpu.SemaphoreType.DMA((NG,)),
            pltpu.SemaphoreType.DMA((NSB,)),
        ],
        compiler_params=pltpu.CompilerParams(
            use_tc_tiling_on_sc=False, needs_layout_passes=False
        ),
    )
    def emb_kernel(idx_hbm, table_hbm, out_hbm, idx_v, rows_v, tout_v, gsem, ssem):
        wid = lax.axis_index("s") * NC + lax.axis_index("c")
        pltpu.sync_copy(idx_hbm.at[:, pl.ds(wid * K, K)], idx_v)

        lane = lax.iota(jnp.int32, L)
        dtv = [lax.shift_right_logical(lane + d0 * L, 3) for d0 in range(D // L)]
        div = [jnp.bitwise_and(lane + d0 * L, DI - 1) for d0 in range(D // L)]

        def transpose_store(ht):
            bg = jnp.bitwise_and(ht, NG - 1)
            bs = jnp.bitwise_and(ht, NSB - 1)

            @pl.when(ht >= NSB)
            def _():
                pltpu.make_async_copy(
                    tout_v.at[bs, :, :, pl.ds(0, K)],
                    out_hbm.at[0, :, wid],
                    ssem.at[bs],
                ).wait()

            pltpu.make_async_copy(
                table_hbm.at[idx_v.at[ht]], rows_v.at[bg], gsem.at[bg]
            ).wait()

            src = rows_v.at[bg]
            dst = tout_v.at[bs]

            @plsc.parallel_loop(0, K, unroll=2)
            def _(b):
                col = jax.lax.broadcast(b, (L,))
                for d0 in range(D // L):
                    vals = src[b, pl.ds(d0 * L, L)]
                    plsc.store_scatter(dst, [dtv[d0], div[d0], col], vals)

            pltpu.async_copy(
                tout_v.at[bs, :, :, pl.ds(0, K)],
                out_hbm.at[ht, :, wid],
                ssem.at[bs],
            )

        def body(h, carry):
            @pl.when(h < H)
            def _():
                bg = jnp.bitwise_and(h, NG - 1)
                pltpu.async_copy(
                    table_hbm.at[idx_v.at[h]], rows_v.at[bg], gsem.at[bg]
                )

            @pl.when(h >= TL)
            def _():
                transpose_store(h - TL)

            return carry

        lax.fori_loop(0, H + TL, body, 0)
        for b in range(NSB):
            pltpu.make_async_copy(
                tout_v.at[b, :, :, pl.ds(0, K)], out_hbm.at[0, :, wid], ssem.at[b]
            ).wait()

    out5 = emb_kernel(xT, embedding)
    return out5.transpose(2, 4, 0, 1, 3).reshape(B, H, D)


# speedup vs baseline: 2.1753x; 1.0049x over previous
import functools

import jax
import jax.numpy as jnp
from jax import lax
from jax.experimental import pallas as pl
from jax.experimental.pallas import tpu as pltpu
from jax.experimental.pallas import tpu_sc as plsc


def kernel(x, embedding):
    B, H = x.shape
    V, D = embedding.shape
    N = B * H

    info = plsc.get_sparse_core_info()
    NC, NS, L = info.num_cores, info.num_subcores, info.num_lanes
    NW = NC * NS

    K = 128
    NG = 8
    TL = 4
    NSB = 4
    DT, DI = D // 8, 8
    assert B == NW * K and D == DT * DI
    RBYTES = K * D * 4

    xT = jnp.swapaxes(x, 0, 1).astype(jnp.int32)

    mesh = plsc.VectorSubcoreMesh(core_axis_name="c", subcore_axis_name="s")

    @functools.partial(
        pl.kernel,
        out_type=jax.ShapeDtypeStruct((H, DT, NW, DI, K), jnp.float32),
        mesh=mesh,
        scratch_types=[
            pltpu.VMEM((H, K), jnp.int32),
            pltpu.VMEM((NG, K, D), jnp.float32),
            pltpu.VMEM((NSB, DT, DI, K + 1), jnp.float32),
            pltpu.SemaphoreType.DMA((NG,)),
            pltpu.SemaphoreType.DMA((NSB,)),
        ],
        compiler_params=pltpu.CompilerParams(
            use_tc_tiling_on_sc=False, needs_layout_passes=False
        ),
    )
    def emb_kernel(idx_hbm, table_hbm, out_hbm, idx_v, rows_v, tout_v, gsem, ssem):
        wid = lax.axis_index("s") * NC + lax.axis_index("c")
        pltpu.sync_copy(idx_hbm.at[:, pl.ds(wid * K, K)], idx_v)

        lane = lax.iota(jnp.int32, L)
        dtv = [lax.shift_right_logical(lane + d0 * L, 3) for d0 in range(D // L)]
        div = [jnp.bitwise_and(lane + d0 * L, DI - 1) for d0 in range(D // L)]

        def transpose_store(ht):
            bg = jnp.bitwise_and(ht, NG - 1)
            bs = jnp.bitwise_and(ht, NSB - 1)

            @pl.when(ht >= NSB)
            def _():
                pltpu.make_async_copy(
                    tout_v.at[bs, :, :, pl.ds(0, K)],
                    out_hbm.at[0, :, wid],
                    ssem.at[bs],
                ).wait()

            pltpu.make_async_copy(
                table_hbm.at[idx_v.at[ht]], rows_v.at[bg], gsem.at[bg]
            ).wait()

            src = rows_v.at[bg]
            dst = tout_v.at[bs]

            @plsc.parallel_loop(0, K, unroll=2)
            def _(b):
                col = jax.lax.broadcast(b, (L,))
                for d0 in range(D // L):
                    vals = src[b, pl.ds(d0 * L, L)]
                    plsc.store_scatter(dst, [dtv[d0], div[d0], col], vals)

            pltpu.async_copy(
                tout_v.at[bs, :, :, pl.ds(0, K)],
                out_hbm.at[ht, :, wid],
                ssem.at[bs],
            )

        def body(h, carry):
            @pl.when(h < H)
            def _():
                bg = jnp.bitwise_and(h, NG - 1)
                pltpu.async_copy(
                    table_hbm.at[idx_v.at[h]], rows_v.at[bg], gsem.at[bg]
                )

            @pl.when(h >= TL)
            def _():
                transpose_store(h - TL)

            return carry

        lax.fori_loop(0, H + TL, body, 0)
        for b in range(NSB):
            pltpu.make_async_copy(
                tout_v.at[b, :, :, pl.ds(0, K)], out_hbm.at[0, :, wid], ssem.at[b]
            ).wait()

    out5 = emb_kernel(xT, embedding)
    return out5.transpose(2, 4, 0, 1, 3).reshape(B, H, D)


# speedup vs baseline: 2.1766x; 1.0006x over previous
import functools

import jax
import jax.numpy as jnp
from jax import lax
from jax.experimental import pallas as pl
from jax.experimental.pallas import tpu as pltpu
from jax.experimental.pallas import tpu_sc as plsc


def kernel(x, embedding):
    B, H = x.shape
    V, D = embedding.shape
    N = B * H

    info = plsc.get_sparse_core_info()
    NC, NS, L = info.num_cores, info.num_subcores, info.num_lanes
    NW = NC * NS

    K = 128
    NG = 8
    TL = 4
    NSB = 4
    DT, DI = D // 8, 8
    assert B == NW * K and D == DT * DI
    RBYTES = K * D * 4

    xT = jnp.swapaxes(x, 0, 1).astype(jnp.int32)

    mesh = plsc.VectorSubcoreMesh(core_axis_name="c", subcore_axis_name="s")

    @functools.partial(
        pl.kernel,
        out_type=jax.ShapeDtypeStruct((H, DT, NW, DI, K), jnp.float32),
        mesh=mesh,
        scratch_types=[
            pltpu.VMEM((H, K), jnp.int32),
            pltpu.VMEM((NG, K, D), jnp.float32),
            pltpu.VMEM((NSB, DT, DI, K + 1), jnp.float32),
            pltpu.SemaphoreType.DMA((NG,)),
            pltpu.SemaphoreType.DMA((NSB,)),
        ],
        compiler_params=pltpu.CompilerParams(
            use_tc_tiling_on_sc=False, needs_layout_passes=False
        ),
    )
    def emb_kernel(idx_hbm, table_hbm, out_hbm, idx_v, rows_v, tout_v, gsem, ssem):
        wid = lax.axis_index("s") * NC + lax.axis_index("c")
        pltpu.sync_copy(idx_hbm.at[:, pl.ds(wid * K, K)], idx_v)

        lane = lax.iota(jnp.int32, L)
        dtv = [lax.shift_right_logical(lane + d0 * L, 3) for d0 in range(D // L)]
        div = [jnp.bitwise_and(lane + d0 * L, DI - 1) for d0 in range(D // L)]

        def transpose_store(ht):
            bg = jnp.bitwise_and(ht, NG - 1)
            bs = jnp.bitwise_and(ht, NSB - 1)

            @pl.when(ht >= NSB)
            def _():
                pltpu.make_async_copy(
                    tout_v.at[bs, :, :, pl.ds(0, K)],
                    out_hbm.at[0, :, wid],
                    ssem.at[bs],
                ).wait()

            pltpu.make_async_copy(
                table_hbm.at[idx_v.at[ht]], rows_v.at[bg], gsem.at[bg]
            ).wait()

            src = rows_v.at[bg]
            dst = tout_v.at[bs]

            @plsc.parallel_loop(0, K, unroll=4)
            def _(b):
                col = jax.lax.broadcast(b, (L,))
                for d0 in range(D // L):
                    vals = src[b, pl.ds(d0 * L, L)]
                    plsc.store_scatter(dst, [dtv[d0], div[d0], col], vals)

            pltpu.async_copy(
                tout_v.at[bs, :, :, pl.ds(0, K)],
                out_hbm.at[ht, :, wid],
                ssem.at[bs],
            )

        def body(h, carry):
            @pl.when(h < H)
            def _():
                bg = jnp.bitwise_and(h, NG - 1)
                pltpu.async_copy(
                    table_hbm.at[idx_v.at[h]], rows_v.at[bg], gsem.at[bg]
                )

            @pl.when(h >= TL)
            def _():
                transpose_store(h - TL)

            return carry

        lax.fori_loop(0, H + TL, body, 0)
        for b in range(NSB):
            pltpu.make_async_copy(
                tout_v.at[b, :, :, pl.ds(0, K)], out_hbm.at[0, :, wid], ssem.at[b]
            ).wait()

    out5 = emb_kernel(xT, embedding)
    return out5.transpose(2, 4, 0, 1, 3).reshape(B, H, D)
